# speedup vs baseline: 5.7619x; 5.7619x over previous
import functools

import jax
import jax.numpy as jnp
from jax import lax
from jax.experimental import pallas as pl
from jax.experimental.pallas import tpu as pltpu
from jax.experimental.pallas import tpu_sc as plsc


def _make_gather(n, v, d, nc, ns):
    nw = nc * ns
    per_w = n // nw
    ch = 128
    n_ch = per_w // ch

    mesh = plsc.VectorSubcoreMesh(core_axis_name="c", subcore_axis_name="s")

    @functools.partial(
        pl.kernel,
        out_type=jax.ShapeDtypeStruct((n, d), jnp.float32),
        mesh=mesh,
        scratch_types=[
            pltpu.VMEM((n_ch, ch), jnp.int32),
            pltpu.VMEM((ch, d), jnp.float32),
            pltpu.SemaphoreType.DMA,
        ],
    )
    def k(ids_hbm, table_hbm, out_hbm, idx_v, rows_v, sem):
        wid = lax.axis_index("s") * nc + lax.axis_index("c")
        base = wid * per_w
        pltpu.sync_copy(ids_hbm.at[wid], idx_v)

        def body(j, carry):
            pltpu.async_copy(table_hbm.at[idx_v.at[j]], rows_v, sem).wait()
            pltpu.sync_copy(rows_v, out_hbm.at[pl.ds(base + j * ch, ch)])
            return carry

        lax.fori_loop(0, n_ch, body, 0)

    return k


def kernel(tok_ids, table):
    b, l = tok_ids.shape
    v, d = table.shape
    n = b * l
    nc, ns = 2, 16
    ids = tok_ids.reshape(nc * ns, n // (nc * ns) // 128, 128)
    out = _make_gather(n, v, d, nc, ns)(ids, table)
    return out.reshape(b, l, d)


# speedup vs baseline: 7.8402x; 1.3607x over previous
import functools

import jax
import jax.numpy as jnp
from jax import lax
from jax.experimental import pallas as pl
from jax.experimental.pallas import tpu as pltpu
from jax.experimental.pallas import tpu_sc as plsc


def _make_gather(n, v, d, nc, ns):
    nw = nc * ns
    per_w = n // nw
    ch = 128
    n_ch = per_w // ch
    nbuf = 5
    assert n_ch % nbuf == 0

    mesh = plsc.VectorSubcoreMesh(core_axis_name="c", subcore_axis_name="s")

    @functools.partial(
        pl.kernel,
        out_type=jax.ShapeDtypeStruct((n, d), jnp.float32),
        mesh=mesh,
        scratch_types=(
            [pltpu.VMEM((n_ch, ch), jnp.int32),
             pltpu.VMEM((nbuf, ch, d), jnp.float32)]
            + [pltpu.SemaphoreType.DMA] * (2 * nbuf)
        ),
    )
    def k(ids_hbm, table_hbm, out_hbm, idx_v, bufs, *sems):
        gsem = sems[:nbuf]
        ssem = sems[nbuf:]
        wid = lax.axis_index("s") * nc + lax.axis_index("c")
        base = wid * per_w
        pltpu.sync_copy(ids_hbm.at[wid], idx_v)

        def gather(b, j):
            pltpu.async_copy(table_hbm.at[idx_v.at[j]], bufs.at[b], gsem[b])

        def store(b, j):
            pltpu.async_copy(bufs.at[b], out_hbm.at[pl.ds(base + j * ch, ch)],
                             ssem[b])

        for b in range(nbuf):
            gather(b, b)

        def body(g, carry):
            c0 = g * nbuf
            for b in range(nbuf):
                pltpu.make_async_copy(
                    table_hbm.at[idx_v.at[c0 + b]], bufs.at[b], gsem[b]).wait()
                store(b, c0 + b)
            for b in range(nbuf):
                pltpu.make_async_copy(
                    bufs.at[b], out_hbm.at[pl.ds(base + (c0 + b) * ch, ch)],
                    ssem[b]).wait()

                @pl.when(c0 + b + nbuf < n_ch)
                def _():
                    gather(b, c0 + b + nbuf)

            return carry

        lax.fori_loop(0, n_ch // nbuf, body, 0)

    return k


def kernel(tok_ids, table):
    b, l = tok_ids.shape
    v, d = table.shape
    n = b * l
    nc, ns = 2, 16
    ids = tok_ids.reshape(nc * ns, n // (nc * ns) // 128, 128)
    out = _make_gather(n, v, d, nc, ns)(ids, table)
    return out.reshape(b, l, d)


# speedup vs baseline: 7.8868x; 1.0059x over previous
import functools

import jax
import jax.numpy as jnp
from jax import lax
from jax.experimental import pallas as pl
from jax.experimental.pallas import tpu as pltpu
from jax.experimental.pallas import tpu_sc as plsc


def _make_gather(n, v, d, nc, ns):
    nw = nc * ns
    per_w = n // nw
    ch = 64
    n_ch = per_w // ch
    nbuf = 10
    assert n_ch % nbuf == 0

    mesh = plsc.VectorSubcoreMesh(core_axis_name="c", subcore_axis_name="s")

    @functools.partial(
        pl.kernel,
        out_type=jax.ShapeDtypeStruct((n, d), jnp.float32),
        mesh=mesh,
        scratch_types=(
            [pltpu.VMEM((n_ch, ch), jnp.int32),
             pltpu.VMEM((nbuf, ch, d), jnp.float32)]
            + [pltpu.SemaphoreType.DMA] * (2 * nbuf)
        ),
    )
    def k(ids_hbm, table_hbm, out_hbm, idx_v, bufs, *sems):
        gsem = sems[:nbuf]
        ssem = sems[nbuf:]
        wid = lax.axis_index("s") * nc + lax.axis_index("c")
        base = wid * per_w
        pltpu.sync_copy(ids_hbm.at[wid], idx_v)

        def gather(b, j):
            pltpu.async_copy(table_hbm.at[idx_v.at[j]], bufs.at[b], gsem[b])

        def store(b, j):
            pltpu.async_copy(bufs.at[b], out_hbm.at[pl.ds(base + j * ch, ch)],
                             ssem[b])

        for b in range(nbuf):
            gather(b, b)

        def body(g, carry):
            c0 = g * nbuf
            for b in range(nbuf):
                pltpu.make_async_copy(
                    table_hbm.at[idx_v.at[c0 + b]], bufs.at[b], gsem[b]).wait()
                store(b, c0 + b)
            for b in range(nbuf):
                pltpu.make_async_copy(
                    bufs.at[b], out_hbm.at[pl.ds(base + (c0 + b) * ch, ch)],
                    ssem[b]).wait()

                @pl.when(c0 + b + nbuf < n_ch)
                def _():
                    gather(b, c0 + b + nbuf)

            return carry

        lax.fori_loop(0, n_ch // nbuf, body, 0)

    return k


def kernel(tok_ids, table):
    b, l = tok_ids.shape
    v, d = table.shape
    n = b * l
    nc, ns = 2, 16
    ids = tok_ids.reshape(nc * ns, n // (nc * ns) // 64, 64)
    out = _make_gather(n, v, d, nc, ns)(ids, table)
    return out.reshape(b, l, d)


# speedup vs baseline: 7.9796x; 1.0118x over previous
import functools

import jax
import jax.numpy as jnp
from jax import lax
from jax.experimental import pallas as pl
from jax.experimental.pallas import tpu as pltpu
from jax.experimental.pallas import tpu_sc as plsc


def _make_gather(n, v, d, nc, ns):
    nw = nc * ns
    per_w = n // nw
    ch = 64
    n_ch = per_w // ch
    nbuf = 10
    npair = nbuf // 2
    assert n_ch % nbuf == 0

    mesh = plsc.VectorSubcoreMesh(core_axis_name="c", subcore_axis_name="s")

    @functools.partial(
        pl.kernel,
        out_type=jax.ShapeDtypeStruct((n, d), jnp.float32),
        mesh=mesh,
        scratch_types=(
            [pltpu.VMEM((n_ch, ch), jnp.int32),
             pltpu.VMEM((npair, 2 * ch, d), jnp.float32)]
            + [pltpu.SemaphoreType.DMA] * (nbuf + npair)
        ),
    )
    def k(ids_hbm, table_hbm, out_hbm, idx_v, bufs, *sems):
        gsem = sems[:nbuf]
        ssem = sems[nbuf:]
        wid = lax.axis_index("s") * nc + lax.axis_index("c")
        base = wid * per_w
        pltpu.sync_copy(ids_hbm.at[wid], idx_v)

        def gather(p, h, j):
            pltpu.async_copy(table_hbm.at[idx_v.at[j]],
                             bufs.at[p, pl.ds(h * ch, ch)], gsem[2 * p + h])

        def gwait(p, h, j):
            pltpu.make_async_copy(table_hbm.at[idx_v.at[j]],
                                  bufs.at[p, pl.ds(h * ch, ch)],
                                  gsem[2 * p + h]).wait()

        def store(p, j):
            pltpu.async_copy(bufs.at[p], out_hbm.at[pl.ds(base + j * ch, 2 * ch)],
                             ssem[p])

        def swait(p, j):
            pltpu.make_async_copy(bufs.at[p],
                                  out_hbm.at[pl.ds(base + j * ch, 2 * ch)],
                                  ssem[p]).wait()

        for p in range(npair):
            for h in range(2):
                gather(p, h, 2 * p + h)

        def body(g, carry):
            c0 = g * nbuf
            for p in range(npair):
                gwait(p, 0, c0 + 2 * p)
                gwait(p, 1, c0 + 2 * p + 1)
                store(p, c0 + 2 * p)
            for p in range(npair):
                swait(p, c0 + 2 * p)
                for h in range(2):
                    @pl.when(c0 + 2 * p + h + nbuf < n_ch)
                    def _():
                        gather(p, h, c0 + 2 * p + h + nbuf)

            return carry

        lax.fori_loop(0, n_ch // nbuf, body, 0)

    return k


def kernel(tok_ids, table):
    b, l = tok_ids.shape
    v, d = table.shape
    n = b * l
    nc, ns = 2, 16
    ids = tok_ids.reshape(nc * ns, n // (nc * ns) // 64, 64)
    out = _make_gather(n, v, d, nc, ns)(ids, table)
    return out.reshape(b, l, d)
